# baseline (speedup 1.0000x reference)
import functools
import math

import jax
import jax.numpy as jnp
from jax import lax
from jax.experimental import pallas as pl
from jax.experimental.pallas import tpu as pltpu
from jax.experimental.pallas import tpu_sc as plsc

B = 16384
NUM_CORES = 2
NUM_SUBCORES = 16
NUM_WORKERS = NUM_CORES * NUM_SUBCORES
B_PER_W = B // NUM_WORKERS
HALF = B_PER_W // 2
LANES = 16
SCALE = math.log(10.0) / 400.0

_mesh = plsc.VectorSubcoreMesh(core_axis_name="c", subcore_axis_name="s")


@functools.partial(
    pl.kernel,
    mesh=_mesh,
    out_type=jax.ShapeDtypeStruct((B,), jnp.float32),
    scratch_types=[
        pltpu.VMEM((2 * B_PER_W,), jnp.int32),
        pltpu.VMEM((2 * B_PER_W,), jnp.float32),
        pltpu.SemaphoreType.DMA,
        pltpu.SemaphoreType.DMA,
        pltpu.SemaphoreType.DMA,
        pltpu.SemaphoreType.DMA,
        pltpu.SemaphoreType.DMA,
    ],
)
def _elo_sc(rating_hbm, home_hbm, away_hbm, out_hbm,
            idx, val, iosem, gsem0, gsem1, gsem2, gsem3):
    wid = lax.axis_index("s") * NUM_CORES + lax.axis_index("c")
    base = wid * B_PER_W
    W = B_PER_W
    hicp = pltpu.async_copy(home_hbm.at[pl.ds(base, W)],
                            idx.at[pl.ds(0, W)], iosem)
    aicp = pltpu.async_copy(away_hbm.at[pl.ds(base, W)],
                            idx.at[pl.ds(W, W)], iosem)
    hicp.wait()
    aicp.wait()
    Q = B_PER_W // 4
    gsems = (gsem0, gsem1, gsem2, gsem3)
    gcps = []
    for q in range(4):
        gcps.append(pltpu.async_copy(
            rating_hbm.at[idx.at[pl.ds(q * Q, Q)]],
            val.at[pl.ds(q * Q, Q)], gsems[q]))
        gcps.append(pltpu.async_copy(
            rating_hbm.at[idx.at[pl.ds(W + q * Q, Q)]],
            val.at[pl.ds(W + q * Q, Q)], gsems[q]))

    def sigmoid_chunk(i):
        sl = pl.ds(i * LANES, LANES)
        x = (val[sl] - val[pl.ds(W + i * LANES, LANES)]) * SCALE
        val[sl] = 1.0 / (1.0 + jnp.exp(x))

    ocps = []
    for q in range(4):
        gcps[2 * q].wait()
        gcps[2 * q + 1].wait()
        plsc.parallel_loop(q * (Q // LANES), (q + 1) * (Q // LANES),
                           unroll=4)(sigmoid_chunk)
        ocps.append(pltpu.async_copy(val.at[pl.ds(q * Q, Q)],
                                     out_hbm.at[pl.ds(base + q * Q, Q)],
                                     iosem))
    for ocp in ocps:
        ocp.wait()


def kernel(rating, home, away):
    return _elo_sc(rating, home.astype(jnp.int32), away.astype(jnp.int32))


# speedup vs baseline: 1.0118x; 1.0118x over previous
import functools
import math

import jax
import jax.numpy as jnp
from jax import lax
from jax.experimental import pallas as pl
from jax.experimental.pallas import tpu as pltpu
from jax.experimental.pallas import tpu_sc as plsc

B = 16384
NUM_CORES = 2
NUM_SUBCORES = 16
NUM_WORKERS = NUM_CORES * NUM_SUBCORES
B_PER_W = B // NUM_WORKERS
HALF = B_PER_W // 2
LANES = 16
SCALE = math.log(10.0) / 400.0

_mesh = plsc.VectorSubcoreMesh(core_axis_name="c", subcore_axis_name="s")


@functools.partial(
    pl.kernel,
    mesh=_mesh,
    out_type=jax.ShapeDtypeStruct((B,), jnp.float32),
    scratch_types=[
        pltpu.VMEM((2 * B_PER_W,), jnp.int32),
        pltpu.VMEM((2 * B_PER_W,), jnp.float32),
        pltpu.SemaphoreType.DMA,
        pltpu.SemaphoreType.DMA,
        pltpu.SemaphoreType.DMA,
    ],
)
def _elo_sc(rating_hbm, home_hbm, away_hbm, out_hbm,
            idx, val, iosem, gsem0, gsem1):
    wid = lax.axis_index("s") * NUM_CORES + lax.axis_index("c")
    base = wid * B_PER_W
    W = B_PER_W
    hicp = pltpu.async_copy(home_hbm.at[pl.ds(base, W)],
                            idx.at[pl.ds(0, W)], iosem)
    aicp = pltpu.async_copy(away_hbm.at[pl.ds(base, W)],
                            idx.at[pl.ds(W, W)], iosem)
    hicp.wait()
    aicp.wait()
    hcp0 = pltpu.async_copy(rating_hbm.at[idx.at[pl.ds(0, HALF)]],
                            val.at[pl.ds(0, HALF)], gsem0)
    acp0 = pltpu.async_copy(rating_hbm.at[idx.at[pl.ds(W, HALF)]],
                            val.at[pl.ds(W, HALF)], gsem0)
    hcp1 = pltpu.async_copy(rating_hbm.at[idx.at[pl.ds(HALF, HALF)]],
                            val.at[pl.ds(HALF, HALF)], gsem1)
    acp1 = pltpu.async_copy(rating_hbm.at[idx.at[pl.ds(W + HALF, HALF)]],
                            val.at[pl.ds(W + HALF, HALF)], gsem1)

    def sigmoid_chunk(i):
        sl = pl.ds(i * LANES, LANES)
        x = (val[sl] - val[pl.ds(W + i * LANES, LANES)]) * SCALE
        val[sl] = 1.0 / (1.0 + jnp.exp(x))

    hcp0.wait()
    acp0.wait()
    plsc.parallel_loop(0, HALF // LANES, unroll=4)(sigmoid_chunk)
    ocp0 = pltpu.async_copy(val.at[pl.ds(0, HALF)],
                            out_hbm.at[pl.ds(base, HALF)], iosem)
    hcp1.wait()
    acp1.wait()
    plsc.parallel_loop(HALF // LANES, B_PER_W // LANES, unroll=4)(sigmoid_chunk)
    ocp1 = pltpu.async_copy(val.at[pl.ds(HALF, HALF)],
                            out_hbm.at[pl.ds(base + HALF, HALF)], iosem)
    ocp0.wait()
    ocp1.wait()


def kernel(rating, home, away):
    return _elo_sc(rating, home.astype(jnp.int32), away.astype(jnp.int32))
